# baseline (speedup 1.0000x reference)
import functools

import jax
import jax.numpy as jnp
from jax import lax
from jax.experimental import pallas as pl
from jax.experimental.pallas import tpu as pltpu
from jax.experimental.pallas import tpu_sc as plsc

D = 64
OW = 128
N_ROWS = 1000000
N_TRIP = 16384
T = 2 * N_TRIP
G = 2 * T

NC, NS = 2, 16
NW = NC * NS
RANGE = 31232
CHUNK_I = 1152
N_CHUNK = 28
LAST_FULL_LO = 998784
TAIL_LO = 999936
TAIL_N = N_ROWS - TAIL_LO
CAP = 4096
SCAN_PIECE = 2048
BLK = 64


def _sc_scan_gather(tblT, idx_flat):
    mesh = plsc.VectorSubcoreMesh(core_axis_name="c", subcore_axis_name="s")

    @functools.partial(
        pl.kernel,
        out_type=jax.ShapeDtypeStruct((G + 1, OW), jnp.float32),
        mesh=mesh,
        scratch_types=[
            pltpu.VMEM((D, CHUNK_I), jnp.float32),
            pltpu.VMEM((D, TAIL_N), jnp.float32),
            pltpu.VMEM((SCAN_PIECE,), jnp.int32),
            pltpu.VMEM((CAP + 16,), jnp.int32),
            pltpu.VMEM((CAP + 16,), jnp.int32),
            pltpu.VMEM((CAP + 16,), jnp.int32),
            pltpu.VMEM((CAP + 16,), jnp.int32),
            pltpu.VMEM((2, BLK, OW), jnp.float32),
            pltpu.VMEM((2, BLK), jnp.int32),
            pltpu.SemaphoreType.DMA,
            pltpu.SemaphoreType.DMA,
        ],
        compiler_params=pltpu.CompilerParams(needs_layout_passes=False),
    )
    def k(tbl_hbm, idx_hbm, out_hbm, staged_v, tail_v, scan_v, hidx_v,
          hpos_v, cidx_v, cpos_v, rows_v, pblk_v, wsem, ssem):
        wid = lax.axis_index("s") * NC + lax.axis_index("c")
        lo = wid * RANGE
        hi = jnp.where(wid == NW - 1, N_ROWS, lo + RANGE)
        iota = lax.iota(jnp.int32, 16)

        def scan_piece(p, wpos):
            pltpu.sync_copy(idx_hbm.at[pl.ds(p * SCAN_PIECE, SCAN_PIECE)],
                            scan_v)

            def scan_vec(s, wpos):
                iv = scan_v[pl.ds(16 * s, 16)]
                pv = iota + (p * SCAN_PIECE + 16 * s)
                m = (iv >= lo) & (iv < hi)
                pref = plsc.cumsum(jnp.where(m, 1, 0))
                posn = wpos + pref - 1
                plsc.store_scatter(hidx_v, [posn], iv, mask=m)
                plsc.store_scatter(hpos_v, [posn], pv, mask=m)
                return jnp.minimum(wpos + pref[15], CAP)

            return lax.fori_loop(0, SCAN_PIECE // 16, scan_vec, wpos)

        n_hits = lax.fori_loop(0, G // SCAN_PIECE, scan_piece, 0)

        qrows = [iota + 16 * q for q in range(D // 16)]

        def process_chunk(src_ref, clo, chi, cmax):
            def cscan(s, cw):
                iv = hidx_v[pl.ds(16 * s, 16)]
                pv = hpos_v[pl.ds(16 * s, 16)]
                valid = (iota + 16 * s) < n_hits
                m = (iv >= clo) & (iv < chi) & valid
                pref = plsc.cumsum(jnp.where(m, 1, 0))
                posn = cw + pref - 1
                plsc.store_scatter(cidx_v, [posn], iv - clo, mask=m)
                plsc.store_scatter(cpos_v, [posn], pv, mask=m)
                return cw + pref[15]

            ccnt = lax.fori_loop(0, (n_hits + 15) // 16, cscan, 0)
            n_blk = (ccnt + BLK - 1) // BLK

            def wait_one():
                pltpu.make_async_copy(
                    rows_v.at[0], out_hbm.at[pblk_v.at[0]], wsem).wait()

            def do_block(b, carry):
                buf = b % 2

                @pl.when(b >= 2)
                def _():
                    wait_one()

                for sb in range(BLK // 16):
                    base = b * BLK + sb * 16
                    xv = cidx_v[pl.ds(base, 16)]
                    xv = jnp.minimum(jnp.maximum(xv, 0), cmax - 1)
                    posv = cpos_v[pl.ds(base, 16)]
                    mvalid = (iota + base) < ccnt
                    posv = jnp.where(mvalid, posv, G)
                    pblk_v[buf, pl.ds(sb * 16, 16)] = posv
                    for l in range(16):
                        col = jnp.broadcast_to(xv[l], (16,))
                        for q in range(D // 16):
                            vals = plsc.load_gather(src_ref, [qrows[q], col])
                            rows_v[buf, sb * 16 + l, pl.ds(16 * q, 16)] = vals
                pltpu.async_copy(rows_v.at[buf], out_hbm.at[pblk_v.at[buf]],
                                 wsem)
                return carry

            lax.fori_loop(0, n_blk, do_block, 0)

            @pl.when(n_blk >= 2)
            def _():
                wait_one()

            @pl.when(n_blk >= 1)
            def _():
                wait_one()

        def stage(dst, clo, width):
            copies = []
            for a in range(D // 8):
                copies.append(pltpu.async_copy(
                    tbl_hbm.at[pl.ds(8 * a, 8), pl.ds(clo, width)],
                    dst.at[pl.ds(8 * a, 8)], ssem))
            for cp in copies:
                cp.wait()

        def chunk_body(c, carry):
            clo = jnp.minimum(lo + c * CHUNK_I, LAST_FULL_LO)
            clo = pl.multiple_of(clo, 128)
            stage(staged_v, clo, CHUNK_I)
            process_chunk(staged_v, clo, clo + CHUNK_I, CHUNK_I)
            return carry

        lax.fori_loop(0, N_CHUNK, chunk_body, 0)
        stage(tail_v, TAIL_LO, TAIL_N)
        process_chunk(tail_v, TAIL_LO, N_ROWS, TAIL_N)

    return k(tblT, idx_flat)


def _tc_finish(rows, pred2, W, b2):
    TBLK = 2048
    n_blk = T // TBLK

    def body(h_ref, t_ref, p_ref, w_ref, b_ref, o_ref):
        pi = pl.program_id(0) // (N_TRIP // TBLK)
        pred = p_ref[pl.ds(pi, 1), :]
        atom = h_ref[:, :D] * t_ref[:, :D] * pred
        o_ref[...] = jnp.dot(atom, w_ref[...],
                             preferred_element_type=jnp.float32) + b_ref[...]

    return pl.pallas_call(
        body,
        grid=(n_blk,),
        in_specs=[
            pl.BlockSpec((TBLK, OW), lambda i: (i, 0)),
            pl.BlockSpec((TBLK, OW), lambda i: (i + n_blk, 0)),
            pl.BlockSpec((2, D), lambda i: (0, 0)),
            pl.BlockSpec((D, D), lambda i: (0, 0)),
            pl.BlockSpec((1, D), lambda i: (0, 0)),
        ],
        out_specs=pl.BlockSpec((TBLK, D), lambda i: (i, 0)),
        out_shape=jax.ShapeDtypeStruct((T, D), jnp.float32),
    )(rows, rows, pred2, W, b2)


def kernel(constant_emb, predicate_emb, W, b, indices_p0, indices_p1):
    idx = jnp.concatenate([indices_p0[:, 0], indices_p1[:, 0],
                           indices_p0[:, 1], indices_p1[:, 1]],
                          axis=0).astype(jnp.int32)
    rows = _sc_scan_gather(constant_emb.T, idx)
    rows = rows[:G]
    pred2 = predicate_emb[:2]
    return _tc_finish(rows, pred2, W, b.reshape(1, D))


# speedup vs baseline: 1.1360x; 1.1360x over previous
import functools

import jax
import jax.numpy as jnp
from jax import lax
from jax.experimental import pallas as pl
from jax.experimental.pallas import tpu as pltpu
from jax.experimental.pallas import tpu_sc as plsc

D = 64
OW = 128
N_ROWS = 1000000
N_TRIP = 16384
T = 2 * N_TRIP
G = 2 * T

NC, NS = 2, 16
NW = NC * NS
RANGE = 31232
CHUNK_I = 1152
N_CHUNK = 28
LAST_FULL_LO = 998784
TAIL_LO = 999936
TAIL_N = N_ROWS - TAIL_LO
CAP = 4096
SCAN_PIECE = 2048
BLK = 64


def _sc_scan_gather(tblT, idx_flat):
    mesh = plsc.VectorSubcoreMesh(core_axis_name="c", subcore_axis_name="s")

    @functools.partial(
        pl.kernel,
        out_type=jax.ShapeDtypeStruct((G + 1, OW), jnp.float32),
        mesh=mesh,
        scratch_types=[
            pltpu.VMEM((D, CHUNK_I), jnp.float32),
            pltpu.VMEM((D, TAIL_N), jnp.float32),
            pltpu.VMEM((SCAN_PIECE,), jnp.int32),
            pltpu.VMEM((CAP + 16,), jnp.int32),
            pltpu.VMEM((CAP + 16,), jnp.int32),
            pltpu.VMEM((CAP + 16,), jnp.int32),
            pltpu.VMEM((CAP + 16,), jnp.int32),
            pltpu.VMEM((2, BLK, OW), jnp.float32),
            pltpu.VMEM((2, BLK), jnp.int32),
            pltpu.SemaphoreType.DMA,
            pltpu.SemaphoreType.DMA,
        ],
        compiler_params=pltpu.CompilerParams(needs_layout_passes=False),
    )
    def k(tbl_hbm, idx_hbm, out_hbm, staged_v, tail_v, scan_v, hidx_v,
          hpos_v, cidx_v, cpos_v, rows_v, pblk_v, wsem, ssem):
        wid = lax.axis_index("s") * NC + lax.axis_index("c")
        lo = wid * RANGE
        hi = jnp.where(wid == NW - 1, N_ROWS, lo + RANGE)
        iota = lax.iota(jnp.int32, 16)

        def scan_piece(p, wpos):
            pltpu.sync_copy(idx_hbm.at[pl.ds(p * SCAN_PIECE, SCAN_PIECE)],
                            scan_v)

            def scan_vec(s, wpos):
                iv = scan_v[pl.ds(16 * s, 16)]
                pv = iota + (p * SCAN_PIECE + 16 * s)
                m = (iv >= lo) & (iv < hi)
                pref = plsc.cumsum(jnp.where(m, 1, 0))
                posn = wpos + pref - 1
                plsc.store_scatter(hidx_v, [posn], iv, mask=m)
                plsc.store_scatter(hpos_v, [posn], pv, mask=m)
                return jnp.minimum(wpos + pref[15], CAP)

            return lax.fori_loop(0, SCAN_PIECE // 16, scan_vec, wpos)

        with jax.named_scope("p1_scan"):
            n_hits = lax.fori_loop(0, G // SCAN_PIECE, scan_piece, 0)

        qrows = [iota + 16 * q for q in range(D // 16)]

        def process_chunk(src_ref, clo, chi, cmax):
            def cscan(s, cw):
                iv = hidx_v[pl.ds(16 * s, 16)]
                pv = hpos_v[pl.ds(16 * s, 16)]
                valid = (iota + 16 * s) < n_hits
                m = (iv >= clo) & (iv < chi) & valid
                pref = plsc.cumsum(jnp.where(m, 1, 0))
                posn = cw + pref - 1
                plsc.store_scatter(cidx_v, [posn], iv - clo, mask=m)
                plsc.store_scatter(cpos_v, [posn], pv, mask=m)
                return cw + pref[15]

            with jax.named_scope("p2_cscan"):
                ccnt = lax.fori_loop(0, (n_hits + 15) // 16, cscan, 0)
            n_blk = (ccnt + BLK - 1) // BLK

            def wait_one():
                pltpu.make_async_copy(
                    rows_v.at[0], out_hbm.at[pblk_v.at[0]], wsem).wait()

            def do_block(b, carry):
                buf = b % 2

                @pl.when(b >= 2)
                def _():
                    wait_one()

                for sb in range(BLK // 16):
                    base = b * BLK + sb * 16
                    xv = cidx_v[pl.ds(base, 16)]
                    xv = jnp.minimum(jnp.maximum(xv, 0), cmax - 1)
                    posv = cpos_v[pl.ds(base, 16)]
                    mvalid = (iota + base) < ccnt
                    posv = jnp.where(mvalid, posv, G)
                    pblk_v[buf, pl.ds(sb * 16, 16)] = posv
                    for l in range(16):
                        col = jnp.broadcast_to(xv[l], (16,))
                        for q in range(D // 16):
                            vals = plsc.load_gather(src_ref, [qrows[q], col])
                            rows_v[buf, sb * 16 + l, pl.ds(16 * q, 16)] = vals
                pltpu.async_copy(rows_v.at[buf], out_hbm.at[pblk_v.at[buf]],
                                 wsem)
                return carry

            with jax.named_scope("p2_gather"):
                lax.fori_loop(0, n_blk, do_block, 0)

            @pl.when(n_blk >= 2)
            def _():
                wait_one()

            @pl.when(n_blk >= 1)
            def _():
                wait_one()

        def stage(dst, clo, width):
            copies = []
            for a in range(D // 8):
                copies.append(pltpu.async_copy(
                    tbl_hbm.at[pl.ds(8 * a, 8), pl.ds(clo, width)],
                    dst.at[pl.ds(8 * a, 8)], ssem))
            for cp in copies:
                cp.wait()

        def chunk_body(c, carry):
            clo = jnp.minimum(lo + c * CHUNK_I, LAST_FULL_LO)
            clo = pl.multiple_of(clo, 128)
            with jax.named_scope("p2_stage"):
                stage(staged_v, clo, CHUNK_I)
            process_chunk(staged_v, clo, clo + CHUNK_I, CHUNK_I)
            plsc.subcore_barrier()
            return carry

        lax.fori_loop(0, N_CHUNK, chunk_body, 0)
        stage(tail_v, TAIL_LO, TAIL_N)
        process_chunk(tail_v, TAIL_LO, N_ROWS, TAIL_N)

    return k(tblT, idx_flat)


def _tc_finish(rows, pred2, W, b2):
    TBLK = 2048
    n_blk = T // TBLK

    def body(h_ref, t_ref, p_ref, w_ref, b_ref, o_ref):
        pi = pl.program_id(0) // (N_TRIP // TBLK)
        pred = p_ref[pl.ds(pi, 1), :]
        atom = h_ref[:, :D] * t_ref[:, :D] * pred
        o_ref[...] = jnp.dot(atom, w_ref[...],
                             preferred_element_type=jnp.float32) + b_ref[...]

    return pl.pallas_call(
        body,
        grid=(n_blk,),
        in_specs=[
            pl.BlockSpec((TBLK, OW), lambda i: (i, 0)),
            pl.BlockSpec((TBLK, OW), lambda i: (i + n_blk, 0)),
            pl.BlockSpec((2, D), lambda i: (0, 0)),
            pl.BlockSpec((D, D), lambda i: (0, 0)),
            pl.BlockSpec((1, D), lambda i: (0, 0)),
        ],
        out_specs=pl.BlockSpec((TBLK, D), lambda i: (i, 0)),
        out_shape=jax.ShapeDtypeStruct((T, D), jnp.float32),
    )(rows, rows, pred2, W, b2)


def kernel(constant_emb, predicate_emb, W, b, indices_p0, indices_p1):
    idx = jnp.concatenate([indices_p0[:, 0], indices_p1[:, 0],
                           indices_p0[:, 1], indices_p1[:, 1]],
                          axis=0).astype(jnp.int32)
    rows = _sc_scan_gather(constant_emb.T, idx)
    rows = rows[:G]
    pred2 = predicate_emb[:2]
    return _tc_finish(rows, pred2, W, b.reshape(1, D))


# speedup vs baseline: 1.4541x; 1.2800x over previous
import functools

import jax
import jax.numpy as jnp
from jax import lax
from jax.experimental import pallas as pl
from jax.experimental.pallas import tpu as pltpu
from jax.experimental.pallas import tpu_sc as plsc

D = 64
H = D // 2
N_TRIP = 16384
T = 2 * N_TRIP

NC, NS = 2, 16
NW = NC * NS
TRIP_PER_W = T // NW
CHUNK = 256
N_CHUNK = TRIP_PER_W // CHUNK
STREAMS = CHUNK // 128
NBUF = 2


def _sc_atom_half(table_h, pred_h, heads2d, tails2d):
    mesh = plsc.VectorSubcoreMesh(core_axis_name="c", subcore_axis_name="s")
    idx_rows = TRIP_PER_W // 128

    @functools.partial(
        pl.kernel,
        out_type=jax.ShapeDtypeStruct((T, H), jnp.float32),
        mesh=mesh,
        scratch_types=[
            pltpu.VMEM((idx_rows, 128), jnp.int32),
            pltpu.VMEM((idx_rows, 128), jnp.int32),
            pltpu.VMEM((H,), jnp.float32),
            pltpu.VMEM((NBUF, CHUNK, H), jnp.float32),
            pltpu.VMEM((NBUF, CHUNK, H), jnp.float32),
            pltpu.VMEM((CHUNK, H), jnp.float32),
            pltpu.SemaphoreType.DMA,
            pltpu.SemaphoreType.DMA,
        ],
        compiler_params=pltpu.CompilerParams(use_tc_tiling_on_sc=False),
    )
    def k(table_hbm, pred_hbm, heads_hbm, tails_hbm, out_hbm,
          hidx_v, tidx_v, pred_v, hrows_v, trows_v, atom_v, hsem, tsem):
        wid = lax.axis_index("s") * NC + lax.axis_index("c")
        base = wid * TRIP_PER_W
        pltpu.sync_copy(heads_hbm.at[pl.ds(wid * idx_rows, idx_rows)], hidx_v)
        pltpu.sync_copy(tails_hbm.at[pl.ds(wid * idx_rows, idx_rows)], tidx_v)
        pltpu.sync_copy(pred_hbm.at[wid // NS], pred_v)
        pk = [pred_v[pl.ds(16 * q, 16)] for q in range(H // 16)]

        def fire(c, buf):
            for j in range(STREAMS):
                pltpu.async_copy(
                    table_hbm.at[hidx_v.at[c * STREAMS + j]],
                    hrows_v.at[buf].at[pl.ds(j * 128, 128)], hsem)
                pltpu.async_copy(
                    table_hbm.at[tidx_v.at[c * STREAMS + j]],
                    trows_v.at[buf].at[pl.ds(j * 128, 128)], tsem)

        def drain(c, buf):
            for j in range(STREAMS):
                pltpu.make_async_copy(
                    table_hbm.at[hidx_v.at[c * STREAMS + j]],
                    hrows_v.at[buf].at[pl.ds(j * 128, 128)], hsem).wait()
                pltpu.make_async_copy(
                    table_hbm.at[tidx_v.at[c * STREAMS + j]],
                    trows_v.at[buf].at[pl.ds(j * 128, 128)], tsem).wait()

        fire(0, 0)
        for c in range(N_CHUNK):
            buf = c % NBUF
            drain(c, buf)
            if c + 1 < N_CHUNK:
                fire(c + 1, (c + 1) % NBUF)

            def rbody(r, carry):
                for q in range(H // 16):
                    sl = pl.ds(16 * q, 16)
                    atom_v[r, sl] = (pk[q] * hrows_v[buf, r, sl]
                                     * trows_v[buf, r, sl])
                return carry

            lax.fori_loop(0, CHUNK, rbody, 0, unroll=4)
            pltpu.sync_copy(atom_v,
                            out_hbm.at[pl.ds(base + c * CHUNK, CHUNK)])

    return k(table_h, pred_h, heads2d, tails2d)


def _tc_matmul(atom_lo, atom_hi, W, b2):
    BLK = 4096
    n_blk = T // BLK

    def body(al_ref, ah_ref, w_ref, b_ref, o_ref):
        atom = jnp.concatenate([al_ref[...], ah_ref[...]], axis=1)
        o_ref[...] = jnp.dot(atom, w_ref[...],
                             preferred_element_type=jnp.float32) + b_ref[...]

    return pl.pallas_call(
        body,
        grid=(n_blk,),
        in_specs=[
            pl.BlockSpec((BLK, H), lambda i: (i, 0)),
            pl.BlockSpec((BLK, H), lambda i: (i, 0)),
            pl.BlockSpec((D, D), lambda i: (0, 0)),
            pl.BlockSpec((1, D), lambda i: (0, 0)),
        ],
        out_specs=pl.BlockSpec((BLK, D), lambda i: (i, 0)),
        out_shape=jax.ShapeDtypeStruct((T, D), jnp.float32),
    )(atom_lo, atom_hi, W, b2)


def kernel(constant_emb, predicate_emb, W, b, indices_p0, indices_p1):
    heads = jnp.concatenate([indices_p0[:, 0], indices_p1[:, 0]], axis=0)
    tails = jnp.concatenate([indices_p0[:, 1], indices_p1[:, 1]], axis=0)
    heads2d = heads.astype(jnp.int32).reshape(T // 128, 128)
    tails2d = tails.astype(jnp.int32).reshape(T // 128, 128)
    tbl_lo = constant_emb[:, :H]
    tbl_hi = constant_emb[:, H:]
    pred_lo = predicate_emb[:2, :H]
    pred_hi = predicate_emb[:2, H:]
    atom_lo = _sc_atom_half(tbl_lo, pred_lo, heads2d, tails2d)
    atom_hi = _sc_atom_half(tbl_hi, pred_hi, heads2d, tails2d)
    return _tc_matmul(atom_lo, atom_hi, W, b.reshape(1, D))


# speedup vs baseline: 2.2968x; 1.5795x over previous
import functools

import jax
import jax.numpy as jnp
from jax import lax
from jax.experimental import pallas as pl
from jax.experimental.pallas import tpu as pltpu
from jax.experimental.pallas import tpu_sc as plsc

D = 64
N_ROWS = 1000000
N_TRIP = 16384
T = 2 * N_TRIP

NC, NS = 2, 16
NW = NC * NS
TRIP_PER_W = T // NW
CHUNK = 256
N_CHUNK = TRIP_PER_W // CHUNK
STREAMS = CHUNK // 128
NBUF = 2


def _sc_atom(table, pred2, heads2d, tails2d):
    mesh = plsc.VectorSubcoreMesh(core_axis_name="c", subcore_axis_name="s")
    idx_rows = TRIP_PER_W // 128

    @functools.partial(
        pl.kernel,
        out_type=jax.ShapeDtypeStruct((T, D), jnp.float32),
        mesh=mesh,
        scratch_types=[
            pltpu.VMEM((idx_rows, 128), jnp.int32),
            pltpu.VMEM((idx_rows, 128), jnp.int32),
            pltpu.VMEM((D,), jnp.float32),
            pltpu.VMEM((NBUF, CHUNK, D), jnp.float32),
            pltpu.VMEM((NBUF, CHUNK, D), jnp.float32),
            pltpu.VMEM((CHUNK, D), jnp.float32),
            pltpu.SemaphoreType.DMA,
            pltpu.SemaphoreType.DMA,
        ],
        compiler_params=pltpu.CompilerParams(use_tc_tiling_on_sc=False),
    )
    def k(table_hbm, pred_hbm, heads_hbm, tails_hbm, out_hbm,
          hidx_v, tidx_v, pred_v, hrows_v, trows_v, atom_v, hsem, tsem):
        wid = lax.axis_index("s") * NC + lax.axis_index("c")
        base = wid * TRIP_PER_W
        pltpu.sync_copy(heads_hbm.at[pl.ds(wid * idx_rows, idx_rows)], hidx_v)
        pltpu.sync_copy(tails_hbm.at[pl.ds(wid * idx_rows, idx_rows)], tidx_v)
        pltpu.sync_copy(pred_hbm.at[wid // NS], pred_v)
        pk = [pred_v[pl.ds(16 * q, 16)] for q in range(D // 16)]

        def fire(c, buf):
            for j in range(STREAMS):
                pltpu.async_copy(
                    table_hbm.at[hidx_v.at[c * STREAMS + j]],
                    hrows_v.at[buf].at[pl.ds(j * 128, 128)], hsem)
                pltpu.async_copy(
                    table_hbm.at[tidx_v.at[c * STREAMS + j]],
                    trows_v.at[buf].at[pl.ds(j * 128, 128)], tsem)

        def drain(c, buf):
            for j in range(STREAMS):
                pltpu.make_async_copy(
                    table_hbm.at[hidx_v.at[c * STREAMS + j]],
                    hrows_v.at[buf].at[pl.ds(j * 128, 128)], hsem).wait()
                pltpu.make_async_copy(
                    table_hbm.at[tidx_v.at[c * STREAMS + j]],
                    trows_v.at[buf].at[pl.ds(j * 128, 128)], tsem).wait()

        fire(0, 0)
        for c in range(N_CHUNK):
            buf = c % NBUF
            drain(c, buf)
            if c + 1 < N_CHUNK:
                fire(c + 1, (c + 1) % NBUF)

            def rbody(r, carry):
                for q in range(D // 16):
                    sl = pl.ds(16 * q, 16)
                    atom_v[r, sl] = (pk[q] * hrows_v[buf, r, sl]
                                     * trows_v[buf, r, sl])
                return carry

            lax.fori_loop(0, CHUNK, rbody, 0, unroll=4)
            pltpu.sync_copy(atom_v,
                            out_hbm.at[pl.ds(base + c * CHUNK, CHUNK)])

    return k(table, pred2, heads2d, tails2d)


def _tc_transpose(tblT):
    BLKC = 2048
    n_blk = (N_ROWS + BLKC - 1) // BLKC

    def body(a_ref, o_ref):
        o_ref[...] = a_ref[...].T

    return pl.pallas_call(
        body,
        grid=(n_blk,),
        in_specs=[pl.BlockSpec((D, BLKC), lambda i: (0, i))],
        out_specs=pl.BlockSpec((BLKC, D), lambda i: (i, 0)),
        out_shape=jax.ShapeDtypeStruct((N_ROWS, D), jnp.float32),
    )(tblT)


def _tc_matmul(atom, W, b2):
    BLK = 4096
    n_blk = T // BLK

    def body(a_ref, w_ref, b_ref, o_ref):
        o_ref[...] = jnp.dot(a_ref[...], w_ref[...],
                             preferred_element_type=jnp.float32) + b_ref[...]

    return pl.pallas_call(
        body,
        grid=(n_blk,),
        in_specs=[
            pl.BlockSpec((BLK, D), lambda i: (i, 0)),
            pl.BlockSpec((D, D), lambda i: (0, 0)),
            pl.BlockSpec((1, D), lambda i: (0, 0)),
        ],
        out_specs=pl.BlockSpec((BLK, D), lambda i: (i, 0)),
        out_shape=jax.ShapeDtypeStruct((T, D), jnp.float32),
    )(atom, W, b2)


def kernel(constant_emb, predicate_emb, W, b, indices_p0, indices_p1):
    heads = jnp.concatenate([indices_p0[:, 0], indices_p1[:, 0]], axis=0)
    tails = jnp.concatenate([indices_p0[:, 1], indices_p1[:, 1]], axis=0)
    heads2d = heads.astype(jnp.int32).reshape(T // 128, 128)
    tails2d = tails.astype(jnp.int32).reshape(T // 128, 128)
    pred2 = predicate_emb[:2]
    table_rm = _tc_transpose(constant_emb.T)
    atom = _sc_atom(table_rm, pred2, heads2d, tails2d)
    return _tc_matmul(atom, W, b.reshape(1, D))


# speedup vs baseline: 3.1420x; 1.3680x over previous
import functools

import jax
import jax.numpy as jnp
from jax import lax
from jax.experimental import pallas as pl
from jax.experimental.pallas import tpu as pltpu
from jax.experimental.pallas import tpu_sc as plsc

D = 64
N_TRIP = 16384
T = 2 * N_TRIP

NC, NS = 2, 16
NW = NC * NS
TRIP_PER_W = T // NW
CHUNK = 256
N_CHUNK = TRIP_PER_W // CHUNK
STREAMS = CHUNK // 128
NBUF = 2


def _sc_atom(table, pred2, heads2d, tails2d):
    mesh = plsc.VectorSubcoreMesh(core_axis_name="c", subcore_axis_name="s")
    idx_rows = TRIP_PER_W // 128

    @functools.partial(
        pl.kernel,
        out_type=jax.ShapeDtypeStruct((T, D), jnp.float32),
        mesh=mesh,
        scratch_types=[
            pltpu.VMEM((idx_rows, 128), jnp.int32),
            pltpu.VMEM((idx_rows, 128), jnp.int32),
            pltpu.VMEM((D,), jnp.float32),
            pltpu.VMEM((NBUF, CHUNK, D), jnp.float32),
            pltpu.VMEM((NBUF, CHUNK, D), jnp.float32),
            pltpu.VMEM((CHUNK, D), jnp.float32),
            pltpu.SemaphoreType.DMA,
            pltpu.SemaphoreType.DMA,
        ],
        compiler_params=pltpu.CompilerParams(use_tc_tiling_on_sc=False),
    )
    def k(table_hbm, pred_hbm, heads_hbm, tails_hbm, out_hbm,
          hidx_v, tidx_v, pred_v, hrows_v, trows_v, atom_v, hsem, tsem):
        wid = lax.axis_index("s") * NC + lax.axis_index("c")
        base = wid * TRIP_PER_W
        pltpu.sync_copy(heads_hbm.at[pl.ds(wid * idx_rows, idx_rows)], hidx_v)
        pltpu.sync_copy(tails_hbm.at[pl.ds(wid * idx_rows, idx_rows)], tidx_v)
        pltpu.sync_copy(pred_hbm.at[wid // NS], pred_v)
        pk = [pred_v[pl.ds(16 * q, 16)] for q in range(D // 16)]

        def fire(c, buf):
            for j in range(STREAMS):
                pltpu.async_copy(
                    table_hbm.at[hidx_v.at[c * STREAMS + j]],
                    hrows_v.at[buf].at[pl.ds(j * 128, 128)], hsem)
                pltpu.async_copy(
                    table_hbm.at[tidx_v.at[c * STREAMS + j]],
                    trows_v.at[buf].at[pl.ds(j * 128, 128)], tsem)

        def drain(c, buf):
            for j in range(STREAMS):
                pltpu.make_async_copy(
                    table_hbm.at[hidx_v.at[c * STREAMS + j]],
                    hrows_v.at[buf].at[pl.ds(j * 128, 128)], hsem).wait()
                pltpu.make_async_copy(
                    table_hbm.at[tidx_v.at[c * STREAMS + j]],
                    trows_v.at[buf].at[pl.ds(j * 128, 128)], tsem).wait()

        fire(0, 0)
        for c in range(N_CHUNK):
            buf = c % NBUF
            drain(c, buf)
            if c + 1 < N_CHUNK:
                fire(c + 1, (c + 1) % NBUF)

            def rbody(r, carry):
                for q in range(D // 16):
                    sl = pl.ds(16 * q, 16)
                    atom_v[r, sl] = (pk[q] * hrows_v[buf, r, sl]
                                     * trows_v[buf, r, sl])
                return carry

            lax.fori_loop(0, CHUNK, rbody, 0, unroll=4)
            pltpu.sync_copy(atom_v,
                            out_hbm.at[pl.ds(base + c * CHUNK, CHUNK)])

    return k(table, pred2, heads2d, tails2d)


def _tc_matmul(atom, W, b2):
    BLK = 4096
    n_blk = T // BLK

    def body(a_ref, w_ref, b_ref, o_ref):
        o_ref[...] = jnp.dot(a_ref[...], w_ref[...],
                             preferred_element_type=jnp.float32) + b_ref[...]

    return pl.pallas_call(
        body,
        grid=(n_blk,),
        in_specs=[
            pl.BlockSpec((BLK, D), lambda i: (i, 0)),
            pl.BlockSpec((D, D), lambda i: (0, 0)),
            pl.BlockSpec((1, D), lambda i: (0, 0)),
        ],
        out_specs=pl.BlockSpec((BLK, D), lambda i: (i, 0)),
        out_shape=jax.ShapeDtypeStruct((T, D), jnp.float32),
    )(atom, W, b2)


def kernel(constant_emb, predicate_emb, W, b, indices_p0, indices_p1):
    heads = jnp.concatenate([indices_p0[:, 0], indices_p1[:, 0]], axis=0)
    tails = jnp.concatenate([indices_p0[:, 1], indices_p1[:, 1]], axis=0)
    heads2d = heads.astype(jnp.int32).reshape(T // 128, 128)
    tails2d = tails.astype(jnp.int32).reshape(T // 128, 128)
    pred2 = predicate_emb[:2]
    atom = _sc_atom(constant_emb, pred2, heads2d, tails2d)
    return _tc_matmul(atom, W, b.reshape(1, D))
